# baseline (speedup 1.0000x reference)
import jax
import jax.numpy as jnp
from jax import lax
from jax.experimental import pallas as pl
from jax.experimental.pallas import tpu as pltpu
from jax.experimental.pallas import tpu_sc as plsc

_RES = 128
_M = _RES * _RES * _RES
_N = 1048576
_NSH = 27
_ROW = 32

_NC = 2
_NS = 16
_NW = _NC * _NS
_PW = _N // _NW
_C = 128
_NCHUNK = _PW // _C
_G = _C // 16

_FB = 4096


def _fmt_body(dens_ref, sh_ref, out_ref):
    out_ref[:, 0:1] = dens_ref[...]
    out_ref[:, 1:1 + _NSH] = sh_ref[...]
    out_ref[:, 1 + _NSH:] = jnp.zeros((_FB, _ROW - 1 - _NSH), jnp.float32)


def _sc_body(coords, table, dens_out, sh_out,
             cc, idxb, wb, rows, densb, shb, sem):
    wid = lax.axis_index("s") * _NC + lax.axis_index("c")
    base0 = wid * _PW

    lane = jnp.arange(16, dtype=jnp.int32)
    lane27 = lane * 27
    maxc = jnp.float32(_RES - 1)

    def chunk_body(c, carry):
        base = base0 + c * _C
        pltpu.sync_copy(coords.at[pl.ds(base, _C)], cc)

        def group_a(g, carry_a):
            p0 = g * 16
            prow = p0 + lane

            def axis_prep(a):
                v = plsc.load_gather(cc, [prow, jnp.full((16,), a, jnp.int32)])
                norm = (v + 1.0) * 0.5
                vox = norm * jnp.float32(_RES)
                vox = jnp.minimum(jnp.maximum(vox, 0.0), maxc)
                i0 = vox.astype(jnp.int32)
                frac = vox - i0.astype(jnp.float32)
                off1 = jnp.minimum(i0 + 1, _RES - 1) - i0
                return i0, off1, frac

            x0, xo, dx = axis_prep(0)
            y0, yo, dy = axis_prep(1)
            z0, zo, dz = axis_prep(2)

            b000 = (z0 * _RES + y0) * _RES + x0
            zoff = zo * (_RES * _RES)
            yoff = yo * _RES
            b100 = b000 + zoff
            b010 = b000 + yoff
            b110 = b100 + yoff
            idxs = (b000, b100, b010, b110,
                    b000 + xo, b100 + xo, b010 + xo, b110 + xo)
            wx0 = 1.0 - dx
            wy0 = 1.0 - dy
            wz0 = 1.0 - dz
            a00 = wx0 * wy0
            a01 = wx0 * dy
            a10 = dx * wy0
            a11 = dx * dy
            ws = (a00 * wz0, a00 * dz, a01 * wz0, a01 * dz,
                  a10 * wz0, a10 * dz, a11 * wz0, a11 * dz)
            for k in range(8):
                idxb[k, pl.ds(p0, 16)] = idxs[k]
                wb[k, pl.ds(p0, 16)] = ws[k]
            return carry_a

        lax.fori_loop(0, _G, group_a, 0)

        descs = []
        for k in range(8):
            descs.append(pltpu.async_copy(
                table.at[idxb.at[k]], rows.at[pl.ds(k * _C, _C)], sem))
        for d in descs:
            d.wait()

        def group_c(g, carry_c):
            p0 = g * 16
            rid0 = p0 + lane
            rids = [rid0 + k * _C for k in range(8)]
            w = [wb[k, pl.ds(p0, 16)] for k in range(8)]
            for j in range(28):
                cid = jnp.full((16,), j, dtype=jnp.int32)
                acc = w[0] * plsc.load_gather(rows, [rids[0], cid])
                for k in range(1, 8):
                    acc = acc + w[k] * plsc.load_gather(rows, [rids[k], cid])
                if j == 0:
                    densb[pl.ds(p0, 16)] = acc
                else:
                    plsc.store_scatter(shb, [lane27 + (27 * p0 + j - 1)], acc)
            return carry_c

        lax.fori_loop(0, _G, group_c, 0)

        pltpu.sync_copy(densb, dens_out.at[pl.ds(base, _C)])
        pltpu.sync_copy(shb, sh_out.at[pl.ds(base * 27, _C * 27)])
        return carry

    lax.fori_loop(0, _NCHUNK, chunk_body, 0)


@jax.jit
def kernel(coords, density, sh_coeffs):
    table = pl.pallas_call(
        _fmt_body,
        grid=(_M // _FB,),
        in_specs=[pl.BlockSpec((_FB, 1), lambda i: (i, 0)),
                  pl.BlockSpec((_FB, _NSH), lambda i: (i, 0))],
        out_specs=pl.BlockSpec((_FB, _ROW), lambda i: (i, 0)),
        out_shape=jax.ShapeDtypeStruct((_M, _ROW), jnp.float32),
    )(density.reshape(_M, 1), sh_coeffs.reshape(_M, _NSH))

    mesh = plsc.VectorSubcoreMesh(core_axis_name="c", subcore_axis_name="s")
    run = pl.kernel(
        _sc_body,
        out_type=(jax.ShapeDtypeStruct((_N,), jnp.float32),
                  jax.ShapeDtypeStruct((_N * _NSH,), jnp.float32)),
        mesh=mesh,
        compiler_params=pltpu.CompilerParams(
            needs_layout_passes=False, use_tc_tiling_on_sc=False),
        scratch_types=[
            pltpu.VMEM((_C, 3), jnp.float32),
            pltpu.VMEM((8, _C), jnp.int32),
            pltpu.VMEM((8, _C), jnp.float32),
            pltpu.VMEM((8 * _C, _ROW), jnp.float32),
            pltpu.VMEM((_C,), jnp.float32),
            pltpu.VMEM((_C * _NSH,), jnp.float32),
            pltpu.SemaphoreType.DMA,
        ],
    )
    dens, sh_flat = run(coords, table)
    return dens, sh_flat.reshape(_N, 3, 9)


# speedup vs baseline: 5.4415x; 5.4415x over previous
import jax
import jax.numpy as jnp
from jax import lax
from jax.experimental import pallas as pl
from jax.experimental.pallas import tpu as pltpu
from jax.experimental.pallas import tpu_sc as plsc

_RES = 128
_PLANE = _RES * _RES
_M = _RES * _PLANE
_N = 1048576
_NSH = 27
_ROW = 32

_NC = 2
_NS = 16
_NW = _NC * _NS

_FV = 1024
_VW = _M // _NW
_FCHUNK = _VW // _FV

_PW = _N // _NW
_C = 128
_NCHUNK = _PW // _C
_G = _C // 16
_SHP = _C + 1


def _fmt_body(dens, planes, table, feat, tout, sem):
    wid = lax.axis_index("s") * _NC + lax.axis_index("c")
    vbase0 = wid * _VW

    lane = jnp.arange(16, dtype=jnp.int32)
    fvecs = [lax.rem(lane + d, jnp.int32(28)) for d in range(28)]

    def chunk_body(i, carry):
        vbase = vbase0 + i * _FV
        z = vbase // _PLANE
        off = vbase - z * _PLANE
        descs = [pltpu.async_copy(dens.at[pl.ds(vbase, _FV)], feat.at[0], sem)]
        for cc3 in range(3):
            for ss9 in range(9):
                src = z * (_NSH * _PLANE) + (cc3 * 9 + ss9) * _PLANE + off
                descs.append(pltpu.async_copy(
                    planes.at[pl.ds(src, _FV)], feat.at[1 + ss9 * 3 + cc3],
                    sem))
        for d in descs:
            d.wait()

        for d in range(28):
            fv = fvecs[d]

            def inner(g, carry_i, fv=fv):
                vrow = g * 16 + lane
                vals = plsc.load_gather(feat, [fv, vrow])
                plsc.store_scatter(tout, [vrow, fv], vals)
                return carry_i

            lax.fori_loop(0, _FV // 16, inner, 0)

        pltpu.sync_copy(tout, table.at[pl.ds(vbase, _FV)])
        return carry

    lax.fori_loop(0, _FCHUNK, chunk_body, 0)


def _take16(vec, idx):
    return lax.gather(
        vec, idx[:, None],
        dimension_numbers=lax.GatherDimensionNumbers(
            offset_dims=(), collapsed_slice_dims=(0,), start_index_map=(0,)),
        slice_sizes=(1,),
        mode=lax.GatherScatterMode.PROMISE_IN_BOUNDS)


def _sc_body(coords, table, dens_out, sh_out,
             cc, idxb, wb, rows, densb, shb, sem):
    wid = lax.axis_index("s") * _NC + lax.axis_index("c")
    base0 = wid * _PW

    lane = jnp.arange(16, dtype=jnp.int32)
    maxc = jnp.float32(_RES - 1)

    def chunk_body(c, carry):
        base = base0 + c * _C
        pltpu.sync_copy(coords.at[pl.ds(base, _C)], cc)

        def group_a(g, carry_a):
            p0 = g * 16
            prow = p0 + lane

            def axis_prep(a):
                v = plsc.load_gather(cc, [prow, jnp.full((16,), a, jnp.int32)])
                norm = (v + 1.0) * 0.5
                vox = norm * jnp.float32(_RES)
                vox = jnp.minimum(jnp.maximum(vox, 0.0), maxc)
                i0 = vox.astype(jnp.int32)
                frac = vox - i0.astype(jnp.float32)
                off1 = jnp.minimum(i0 + 1, _RES - 1) - i0
                return i0, off1, frac

            x0, xo, dx = axis_prep(0)
            y0, yo, dy = axis_prep(1)
            z0, zo, dz = axis_prep(2)

            b000 = (z0 * _RES + y0) * _RES + x0
            zoff = zo * _PLANE
            yoff = yo * _RES
            b100 = b000 + zoff
            b010 = b000 + yoff
            b110 = b100 + yoff
            idxs = (b000, b100, b010, b110,
                    b000 + xo, b100 + xo, b010 + xo, b110 + xo)
            wx0 = 1.0 - dx
            wy0 = 1.0 - dy
            wz0 = 1.0 - dz
            a00 = wx0 * wy0
            a01 = wx0 * dy
            a10 = dx * wy0
            a11 = dx * dy
            ws = (a00 * wz0, a00 * dz, a01 * wz0, a01 * dz,
                  a10 * wz0, a10 * dz, a11 * wz0, a11 * dz)
            for k in range(8):
                idxb[k, pl.ds(p0, 16)] = idxs[k]
                wb[k, pl.ds(p0, 16)] = ws[k]
            return carry_a

        lax.fori_loop(0, _G, group_a, 0)

        descs = []
        for k in range(8):
            descs.append(pltpu.async_copy(
                table.at[idxb.at[k]], rows.at[pl.ds(k * _C, _C)], sem))
        for d in descs:
            d.wait()

        def group_c(g, carry_c):
            p0 = g * 16
            w_vecs = [wb[k, pl.ds(p0, 16)] for k in range(8)]
            for q in range(16):
                p = p0 + q
                sel = jnp.full((16,), q, dtype=jnp.int32)
                acc0 = jnp.zeros((16,), jnp.float32)
                acc1 = jnp.zeros((16,), jnp.float32)
                for k in range(8):
                    wk = _take16(w_vecs[k], sel)
                    r = k * _C + p
                    acc0 = acc0 + wk * rows[r, pl.ds(0, 16)]
                    acc1 = acc1 + wk * rows[r, pl.ds(16, 16)]
                plsc.store_scatter(
                    densb, [jnp.full((16,), p, dtype=jnp.int32)], acc0,
                    mask=lane == 0)
                pvec = jnp.full((16,), p, dtype=jnp.int32)
                plsc.store_scatter(
                    shb, [lane - 1, pvec], acc0, mask=lane >= 1)
                plsc.store_scatter(
                    shb, [lane + 15, pvec], acc1, mask=lane < 12)
            return carry_c

        lax.fori_loop(0, _G, group_c, 0)

        pltpu.sync_copy(densb, dens_out.at[pl.ds(base, _C)])
        pltpu.sync_copy(shb.at[:, pl.ds(0, _C)],
                        sh_out.at[:, pl.ds(base, _C)])
        return carry

    lax.fori_loop(0, _NCHUNK, chunk_body, 0)


@jax.jit
def kernel(coords, density, sh_coeffs):
    dens_flat = density.reshape(_M)
    planes = jnp.transpose(sh_coeffs, (0, 3, 4, 1, 2)).reshape(
        _RES * 3 * 9 * _PLANE)

    mesh = plsc.VectorSubcoreMesh(core_axis_name="c", subcore_axis_name="s")
    params = pltpu.CompilerParams(
        needs_layout_passes=False, use_tc_tiling_on_sc=False)

    table = pl.kernel(
        _fmt_body,
        out_type=jax.ShapeDtypeStruct((_M, _ROW), jnp.float32),
        mesh=mesh,
        compiler_params=params,
        scratch_types=[
            pltpu.VMEM((28, _FV), jnp.float32),
            pltpu.VMEM((_FV, _ROW), jnp.float32),
            pltpu.SemaphoreType.DMA,
        ],
    )(dens_flat, planes)

    run = pl.kernel(
        _sc_body,
        out_type=(jax.ShapeDtypeStruct((_N,), jnp.float32),
                  jax.ShapeDtypeStruct((_NSH, _N), jnp.float32)),
        mesh=mesh,
        compiler_params=params,
        scratch_types=[
            pltpu.VMEM((_C, 3), jnp.float32),
            pltpu.VMEM((8, _C), jnp.int32),
            pltpu.VMEM((8, _C), jnp.float32),
            pltpu.VMEM((8 * _C, _ROW), jnp.float32),
            pltpu.VMEM((_C,), jnp.float32),
            pltpu.VMEM((_NSH, _SHP), jnp.float32),
            pltpu.SemaphoreType.DMA,
        ],
    )
    dens, sh27 = run(coords, table)
    return dens, jnp.transpose(sh27.reshape(9, 3, _N), (2, 1, 0))
